# baseline (speedup 1.0000x reference)
import jax
import jax.numpy as jnp
from jax.experimental import pallas as pl
from jax.experimental.pallas import tpu as pltpu

_CH = 1024
_NBUF = 3
_NSPLIT = 4
_SUB = _CH // _NSPLIT


def _router_body(x_hbm, w1_ref, b1_ref, w2_ref, b2_ref, o_ref, buf, sem,
                 obuf, osem):
    n_chunks = x_hbm.shape[0] // _CH

    def _out_copy(chunk, slot):
        return pltpu.make_async_copy(
            obuf.at[slot],
            o_ref.at[pl.ds(chunk * _CH, _CH), :],
            osem.at[slot],
        )

    def _sub_copy(chunk, slot, j):
        return pltpu.make_async_copy(
            x_hbm.at[pl.ds(chunk * _CH + j * _SUB, _SUB), :],
            buf.at[slot, pl.ds(j * _SUB, _SUB), :],
            sem.at[slot, j],
        )

    def _copy_in(chunk, slot):
        for j in range(_NSPLIT):
            _sub_copy(chunk, slot, j).start()

    for slot in range(min(_NBUF, n_chunks)):
        _copy_in(slot, slot)

    def _step(i, carry):
        slot = jax.lax.rem(i, _NBUF)
        for j in range(_NSPLIT):
            _sub_copy(i, slot, j).wait()
        x = buf[slot].astype(jnp.bfloat16)
        h = jnp.dot(x, w1_ref[...], preferred_element_type=jnp.float32)
        h = jnp.maximum(h + b1_ref[...], 0.0)
        logits = jnp.dot(h, w2_ref[...], preferred_element_type=jnp.float32)
        logits = logits + b2_ref[...]
        m = jnp.max(logits, axis=1, keepdims=True)
        e = jnp.exp(logits - m)

        @pl.when(i >= _NBUF)
        def _():
            _out_copy(i - _NBUF, slot).wait()

        obuf[slot] = e / jnp.sum(e, axis=1, keepdims=True)
        _out_copy(i, slot).start()

        @pl.when(i + _NBUF < n_chunks)
        def _():
            _copy_in(i + _NBUF, slot)

        return carry

    jax.lax.fori_loop(0, n_chunks, _step, 0)

    for c in range(max(0, n_chunks - _NBUF), n_chunks):
        _out_copy(c, c % _NBUF).wait()


def kernel(prnet_features, W1, b1, W2, b2):
    n, d = prnet_features.shape
    hidden = W1.shape[1]
    ne = W2.shape[1]
    return pl.pallas_call(
        _router_body,
        in_specs=[
            pl.BlockSpec(memory_space=pltpu.MemorySpace.HBM),
            pl.BlockSpec(memory_space=pltpu.MemorySpace.VMEM),
            pl.BlockSpec(memory_space=pltpu.MemorySpace.VMEM),
            pl.BlockSpec(memory_space=pltpu.MemorySpace.VMEM),
            pl.BlockSpec(memory_space=pltpu.MemorySpace.VMEM),
        ],
        out_specs=pl.BlockSpec(memory_space=pltpu.MemorySpace.HBM),
        out_shape=jax.ShapeDtypeStruct((n, ne), jnp.float32),
        scratch_shapes=[
            pltpu.VMEM((_NBUF, _CH, d), jnp.float32),
            pltpu.SemaphoreType.DMA((_NBUF, _NSPLIT)),
            pltpu.VMEM((_NBUF, _CH, 64), jnp.float32),
            pltpu.SemaphoreType.DMA((_NBUF,)),
        ],
        compiler_params=pltpu.CompilerParams(
            vmem_limit_bytes=60 * 1024 * 1024,
        ),
    )(prnet_features, W1.astype(jnp.bfloat16), b1.reshape(1, hidden),
      W2, b2.reshape(1, ne))


# speedup vs baseline: 1.0083x; 1.0083x over previous
import jax
import jax.numpy as jnp
from jax.experimental import pallas as pl
from jax.experimental.pallas import tpu as pltpu

_BM = 512


def _router_body(x_hbm, w1_ref, b1_ref, w2_ref, b2_ref, o_hbm):
    n, d = x_hbm.shape
    ne = o_hbm.shape[1]

    def _inner(x_ref, o_ref):
        x = x_ref[...].astype(jnp.bfloat16)
        h = jnp.dot(x, w1_ref[...], preferred_element_type=jnp.float32)
        h = jnp.maximum(h + b1_ref[...], 0.0)
        logits = jnp.dot(h, w2_ref[...], preferred_element_type=jnp.float32)
        logits = logits + b2_ref[...]
        m = jnp.max(logits, axis=1, keepdims=True)
        e = jnp.exp(logits - m)
        o_ref[...] = e / jnp.sum(e, axis=1, keepdims=True)

    pipe = pltpu.emit_pipeline(
        _inner,
        grid=(n // _BM,),
        in_specs=[
            pl.BlockSpec((_BM, d), lambda i: (i, 0),
                         pipeline_mode=pl.Buffered(buffer_count=4)),
        ],
        out_specs=[
            pl.BlockSpec((_BM, ne), lambda i: (i, 0)),
        ],
    )
    pipe(x_hbm, o_hbm)


def kernel(prnet_features, W1, b1, W2, b2):
    n, d = prnet_features.shape
    hidden = W1.shape[1]
    ne = W2.shape[1]
    return pl.pallas_call(
        _router_body,
        in_specs=[
            pl.BlockSpec(memory_space=pltpu.MemorySpace.HBM),
            pl.BlockSpec(memory_space=pltpu.MemorySpace.VMEM),
            pl.BlockSpec(memory_space=pltpu.MemorySpace.VMEM),
            pl.BlockSpec(memory_space=pltpu.MemorySpace.VMEM),
            pl.BlockSpec(memory_space=pltpu.MemorySpace.VMEM),
        ],
        out_specs=pl.BlockSpec(memory_space=pltpu.MemorySpace.HBM),
        out_shape=jax.ShapeDtypeStruct((n, ne), jnp.float32),
        compiler_params=pltpu.CompilerParams(
            vmem_limit_bytes=63 * 1024 * 1024,
        ),
    )(prnet_features, W1.astype(jnp.bfloat16), b1.reshape(1, hidden),
      W2, b2.reshape(1, ne))


# speedup vs baseline: 1.0199x; 1.0116x over previous
import jax
import jax.numpy as jnp
from jax.experimental import pallas as pl
from jax.experimental.pallas import tpu as pltpu

_BM = 256


def _router_body(x_hbm, w1_ref, b1_ref, w2_ref, b2_ref, o_hbm):
    n, d = x_hbm.shape
    ne = o_hbm.shape[1]

    def _inner(x_ref, o_ref):
        x = x_ref[...].astype(jnp.bfloat16)
        h = jnp.dot(x, w1_ref[...], preferred_element_type=jnp.float32)
        h = jnp.maximum(h + b1_ref[...], 0.0)
        logits = jnp.dot(h, w2_ref[...], preferred_element_type=jnp.float32)
        logits = logits + b2_ref[...]
        m = jnp.max(logits, axis=1, keepdims=True)
        e = jnp.exp(logits - m)
        o_ref[...] = e / jnp.sum(e, axis=1, keepdims=True)

    pipe = pltpu.emit_pipeline(
        _inner,
        grid=(n // _BM,),
        in_specs=[
            pl.BlockSpec((_BM, d), lambda i: (i, 0),
                         pipeline_mode=pl.Buffered(buffer_count=8)),
        ],
        out_specs=[
            pl.BlockSpec((_BM, ne), lambda i: (i, 0)),
        ],
    )
    pipe(x_hbm, o_hbm)


def kernel(prnet_features, W1, b1, W2, b2):
    n, d = prnet_features.shape
    hidden = W1.shape[1]
    ne = W2.shape[1]
    return pl.pallas_call(
        _router_body,
        in_specs=[
            pl.BlockSpec(memory_space=pltpu.MemorySpace.HBM),
            pl.BlockSpec(memory_space=pltpu.MemorySpace.VMEM),
            pl.BlockSpec(memory_space=pltpu.MemorySpace.VMEM),
            pl.BlockSpec(memory_space=pltpu.MemorySpace.VMEM),
            pl.BlockSpec(memory_space=pltpu.MemorySpace.VMEM),
        ],
        out_specs=pl.BlockSpec(memory_space=pltpu.MemorySpace.HBM),
        out_shape=jax.ShapeDtypeStruct((n, ne), jnp.float32),
        compiler_params=pltpu.CompilerParams(
            vmem_limit_bytes=63 * 1024 * 1024,
        ),
    )(prnet_features, W1.astype(jnp.bfloat16), b1.reshape(1, hidden),
      W2, b2.reshape(1, ne))
